# baseline (speedup 1.0000x reference)
import functools

import jax
import jax.numpy as jnp
from jax.experimental import pallas as pl
from jax.experimental.pallas import tpu as pltpu

N_LAYERS = 6
HID = 64


def _proj_body(h_ref, w_ref, o_ref):
    o_ref[...] = jnp.dot(h_ref[...], w_ref[...],
                         preferred_element_type=jnp.float32)


@functools.partial(jax.jit, static_argnames=("block_rows",))
def _proj(h, w, block_rows=2000):
    n, d = h.shape
    _, dout = w.shape
    grid = (n // block_rows,)
    return pl.pallas_call(
        _proj_body,
        grid=grid,
        in_specs=[
            pl.BlockSpec((block_rows, d), lambda i: (i, 0)),
            pl.BlockSpec((d, dout), lambda i: (0, 0)),
        ],
        out_specs=pl.BlockSpec((block_rows, dout), lambda i: (i, 0)),
        out_shape=jax.ShapeDtypeStruct((n, dout), jnp.float32),
    )(h, w)


def _gatv2_layer(h, src, dst, n, lp):
    d_in = h.shape[1]
    w = jnp.concatenate([lp['Wl'], lp['Wr']], axis=1)
    if d_in % 8 != 0:
        pad = 8 - d_in % 8
        h = jnp.pad(h, ((0, 0), (0, pad)))
        w = jnp.pad(w, ((0, pad), (0, 0)))
    xlr = _proj(h, w)
    xl, xr = xlr[:, :HID], xlr[:, HID:]
    e = jax.nn.leaky_relu(xl[src] + xr[dst], 0.2) @ lp['att']
    m = jax.ops.segment_max(e, dst, num_segments=n)
    ex = jnp.exp(e - m[dst])
    s = jax.ops.segment_sum(ex, dst, num_segments=n)
    alpha = ex / (s[dst] + 1e-16)
    out = jax.ops.segment_sum(xl[src] * alpha[:, None], dst, num_segments=n)
    return out + lp['b']


def kernel(x, edge_index, batch, protein, params):
    n = x.shape[0]
    b = protein.shape[0]
    idx = x[:, 0].astype(jnp.int32)
    q = params['emb'][idx]
    h = jnp.concatenate([x, q], axis=1)
    loops = jnp.arange(n, dtype=edge_index.dtype)
    ei = jnp.concatenate([edge_index, jnp.stack([loops, loops])], axis=1)
    src, dst = ei[0], ei[1]
    for i, lp in enumerate(params['gat']):
        h = _gatv2_layer(h, src, dst, n, lp)
        if i < N_LAYERS - 1:
            h = jax.nn.leaky_relu(h, 0.01)
    counts = jax.ops.segment_sum(jnp.ones((n,), jnp.float32), batch, num_segments=b)
    p1 = jax.ops.segment_sum(h, batch, num_segments=b) / jnp.maximum(counts, 1.0)[:, None]
    p2 = jax.ops.segment_max(h, batch, num_segments=b)
    z = jnp.concatenate([p1, protein, p2], axis=1)
    z = jax.nn.leaky_relu(z @ params['fcW'] + params['fcb'], 0.01)
    out = z @ params['fc2W'] + params['fc2b']
    return out
